# baseline (speedup 1.0000x reference)
import jax
import jax.numpy as jnp
from jax.experimental import pallas as pl
from jax.experimental.pallas import tpu as pltpu

B, H, KV, HD = 64, 32, 8, 128
BS, MAXC = 16, 2048
MB = MAXC // BS
G = H // KV
SCALE = HD ** -0.5
BPI = 8
T = BPI * BS
LANES = BPI * BS * KV
CPI = 3
NEG = -1e30


def _attn_kernel(phys_ref, ctx_ref, q_ref, kn_ref, vn_ref, kc_hbm, vc_hbm,
                 out_ref, kbuf, vbuf, hmask, tokm, ksem, vsem):
    b = pl.program_id(0)
    ctx = ctx_ref[b]
    nc = (ctx - 1 + T - 1) // T

    @pl.when(b == 0)
    def _mask_init():
        col = jax.lax.broadcasted_iota(jnp.int32, (H, LANES), 1)
        row = jax.lax.broadcasted_iota(jnp.int32, (H, LANES), 0)
        match = (col & (KV - 1)) == (row >> 2)
        hmask[...] = jnp.where(match, 0.0, NEG)
        tokm[...] = col >> 3
        vbuf[...] = jnp.zeros_like(vbuf)

    def issue(c, slot):
        base = c * BPI
        for i in range(BPI):
            pid = phys_ref[b, base + i]
            pltpu.make_async_copy(kc_hbm.at[pid], kbuf.at[slot, i],
                                  ksem.at[slot]).start()
            pltpu.make_async_copy(vc_hbm.at[pid], vbuf.at[slot, i],
                                  vsem.at[slot]).start()

    issue(0, 0)
    for _c in range(1, CPI):
        @pl.when(_c < nc)
        def _issue_first(_c=_c):
            issue(_c, _c)

    q = q_ref[0]
    hmask_v = hmask[...]
    tok = tokm[...]

    def wait_slot(slot):
        pltpu.make_async_copy(kc_hbm.at[0], kbuf.at[slot], ksem.at[slot]).wait()
        pltpu.make_async_copy(vc_hbm.at[0], vbuf.at[slot], vsem.at[slot]).wait()

    def chunk_update(c, slot, carry):
        m_p, l_p, acc_p = carry
        k2 = kbuf[slot].reshape(LANES, HD)
        s = jax.lax.dot_general(
            q, k2, (((1,), (1,)), ((), ())),
            preferred_element_type=jnp.float32) + hmask_v
        s = jnp.where(tok < ctx - 1 - c * T, s, NEG)

        m_n = jnp.maximum(m_p, jnp.max(s, axis=1, keepdims=True))
        corr = jnp.exp(m_p - m_n)
        p = jnp.exp(s - m_n[:, :1])
        l_n = l_p * corr + jnp.sum(p, axis=1, keepdims=True)

        v2 = vbuf[slot].reshape(LANES, HD)
        pv = jax.lax.dot_general(
            p, v2, (((1,), (0,)), ((), ())),
            preferred_element_type=jnp.float32)
        acc_n = acc_p * corr + pv
        return (m_n, l_n, acc_n)

    def body(i, carry):
        c0 = CPI * i
        for d in range(CPI):
            @pl.when(c0 + d + CPI < nc)
            def _prefetch(d=d):
                c = c0 + d + CPI
                issue(c, jax.lax.rem(c, 2 * CPI))

        wait_slot(jax.lax.rem(c0, 2 * CPI))
        for d in range(1, CPI):
            @pl.when(c0 + d < nc)
            def _wait(d=d):
                wait_slot(jax.lax.rem(c0 + d, 2 * CPI))

        for d in range(CPI):
            carry = chunk_update(c0 + d, jax.lax.rem(c0 + d, 2 * CPI), carry)
        return carry

    m0 = jnp.full((H, 128), NEG, jnp.float32)
    l0 = jnp.zeros((H, 128), jnp.float32)
    a0 = jnp.zeros((H, HD), jnp.float32)
    m_f, l_f, acc_f = jax.lax.fori_loop(0, (nc + CPI - 1) // CPI,
                                        body, (m0, l0, a0))

    s_new = jax.lax.dot_general(
        q, kn_ref[0], (((1,), (1,)), ((), ())),
        preferred_element_type=jnp.float32)
    col8 = jax.lax.broadcasted_iota(jnp.int32, (H, KV), 1)
    row8 = jax.lax.broadcasted_iota(jnp.int32, (H, KV), 0)
    sn = jnp.where(col8 == (row8 >> 2), s_new, NEG)
    m2 = jnp.maximum(m_f, jnp.max(sn, axis=1, keepdims=True))
    corr = jnp.exp(m_f - m2)
    pn = jnp.sum(jnp.exp(sn - m2[:, :1]), axis=1, keepdims=True)
    l2 = l_f * corr + pn
    vn_rep = jnp.repeat(vn_ref[0], G, axis=0)
    out_ref[0, 0] = (acc_f * corr + pn * vn_rep) / l2


def _paged_attn(q, k, v, k_cache, v_cache, block_tables, context_lens,
                interpret=False):
    last_page = (context_lens - 1) // BS
    page_pos = jnp.minimum(jnp.arange(MB, dtype=jnp.int32)[None, :],
                           last_page[:, None])
    phys = jnp.take_along_axis(block_tables, page_pos, axis=1)

    grid_spec = pltpu.PrefetchScalarGridSpec(
        num_scalar_prefetch=2,
        grid=(B,),
        in_specs=[
            pl.BlockSpec((1, H, HD), lambda b, phys, ctx: (b, 0, 0)),
            pl.BlockSpec((1, KV, HD), lambda b, phys, ctx: (b, 0, 0)),
            pl.BlockSpec((1, KV, HD), lambda b, phys, ctx: (b, 0, 0)),
            pl.BlockSpec(memory_space=pl.ANY),
            pl.BlockSpec(memory_space=pl.ANY),
        ],
        out_specs=pl.BlockSpec((1, 1, H, HD),
                               lambda b, phys, ctx: (b, 0, 0, 0)),
        scratch_shapes=[
            pltpu.VMEM((2 * CPI, BPI, BS, KV, HD), jnp.float32),
            pltpu.VMEM((2 * CPI, BPI, BS, KV, HD), jnp.float32),
            pltpu.VMEM((H, LANES), jnp.float32),
            pltpu.VMEM((H, LANES), jnp.int32),
            pltpu.SemaphoreType.DMA((2 * CPI,)),
            pltpu.SemaphoreType.DMA((2 * CPI,)),
        ],
    )
    out = pl.pallas_call(
        _attn_kernel,
        grid_spec=grid_spec,
        out_shape=jax.ShapeDtypeStruct((B, 1, H, HD), jnp.float32),
        compiler_params=pltpu.CompilerParams(
            dimension_semantics=("arbitrary",),
        ),
        name="paged_decode_attn",
        interpret=interpret,
    )(phys, context_lens, q * SCALE, k, v, k_cache, v_cache)
    return out


def kernel(q, k, v, k_cache, v_cache, slot_mapping, block_tables, context_lens):
    del slot_mapping
    return _paged_attn(q, k, v, k_cache, v_cache, block_tables, context_lens)


# speedup vs baseline: 1.2033x; 1.2033x over previous
import jax
import jax.numpy as jnp
from jax.experimental import pallas as pl
from jax.experimental.pallas import tpu as pltpu

B, H, KV, HD = 64, 32, 8, 128
BS, MAXC = 16, 2048
MB = MAXC // BS
G = H // KV
SCALE = HD ** -0.5
BPI = 8
T = BPI * BS
LANES = BPI * BS * KV
CPI = 4
NEG = -1e30


def _attn_kernel(phys_ref, ctx_ref, q_ref, kn_ref, vn_ref, kc_hbm, vc_hbm,
                 out_ref, kbuf, vbuf, hmask, tokm, ksem, vsem):
    b = pl.program_id(0)
    ctx = ctx_ref[b]
    nfull = (ctx - 1) // T
    nc = (ctx - 1 + T - 1) // T

    def slot_of(c):
        return jnp.where(c == 0, 8 + jax.lax.rem(b, 2), jax.lax.rem(c, 8))

    @pl.when(b == 0)
    def _mask_init():
        col = jax.lax.broadcasted_iota(jnp.int32, (H, LANES), 1)
        row = jax.lax.broadcasted_iota(jnp.int32, (H, LANES), 0)
        match = (col & (KV - 1)) == (row >> 2)
        hmask[...] = jnp.where(match, 0.0, NEG)
        tokm[...] = col >> 3
        vbuf[...] = jnp.zeros_like(vbuf)

    def issue(c, slot, row=None):
        base = c * BPI
        for i in range(BPI):
            pid = phys_ref[b if row is None else row, base + i]
            pltpu.make_async_copy(kc_hbm.at[pid], kbuf.at[slot, i],
                                  ksem.at[slot]).start()
            pltpu.make_async_copy(vc_hbm.at[pid], vbuf.at[slot, i],
                                  vsem.at[slot]).start()

    @pl.when(b == 0)
    def _issue0():
        issue(0, 8)
    for _c in range(1, CPI):
        @pl.when(_c < nc)
        def _issue_first(_c=_c):
            issue(_c, _c)

    @pl.when(b + 1 < B)
    def _issue_next_b():
        issue(0, 8 + jax.lax.rem(b + 1, 2), row=b + 1)

    q = q_ref[0]
    hmask_v = hmask[...]
    tok = tokm[...]

    def wait_slot(slot):
        pltpu.make_async_copy(kc_hbm.at[0], kbuf.at[slot], ksem.at[slot]).wait()
        pltpu.make_async_copy(vc_hbm.at[0], vbuf.at[slot], vsem.at[slot]).wait()

    def chunk_update(slot, carry, c=None):
        m_p, l_p, acc_p = carry
        k2 = kbuf[slot].reshape(LANES, HD)
        s = jax.lax.dot_general(
            q, k2, (((1,), (1,)), ((), ())),
            preferred_element_type=jnp.float32) + hmask_v
        if c is not None:
            s = jnp.where(tok < ctx - 1 - c * T, s, NEG)

        m_n = jnp.maximum(m_p, jnp.max(s, axis=1, keepdims=True))
        corr = jnp.exp(m_p - m_n)
        p = jnp.exp(s - m_n[:, :1])
        l_n = l_p * corr + jnp.sum(p, axis=1, keepdims=True)

        v2 = vbuf[slot].reshape(LANES, HD)
        pv = jax.lax.dot_general(
            p, v2, (((1,), (0,)), ((), ())),
            preferred_element_type=jnp.float32)
        acc_n = acc_p * corr + pv
        return (m_n, l_n, acc_n)

    def body(i, carry):
        c0 = CPI * i
        for d in range(CPI):
            @pl.when(c0 + d + CPI < nc)
            def _prefetch(d=d):
                c = c0 + d + CPI
                issue(c, jax.lax.rem(c, 2 * CPI))

        for d in range(CPI):
            wait_slot(slot_of(c0 + d))
        for d in range(CPI):
            carry = chunk_update(slot_of(c0 + d), carry)
        return carry

    def body1(c, carry):
        wait_slot(slot_of(c))
        return chunk_update(slot_of(c), carry)

    m0 = jnp.full((H, 128), NEG, jnp.float32)
    l0 = jnp.zeros((H, 128), jnp.float32)
    a0 = jnp.zeros((H, HD), jnp.float32)
    nf_main = (nfull // CPI) * CPI
    carry = jax.lax.fori_loop(0, nfull // CPI, body, (m0, l0, a0))
    carry = jax.lax.fori_loop(nf_main, nfull, body1, carry)

    @pl.when(nc > nfull)
    def _wait_partial():
        wait_slot(slot_of(nfull))
    m_f, l_f, acc_f = chunk_update(slot_of(nfull), carry, c=nfull)

    s_new = jax.lax.dot_general(
        q, kn_ref[0], (((1,), (1,)), ((), ())),
        preferred_element_type=jnp.float32)
    col8 = jax.lax.broadcasted_iota(jnp.int32, (H, KV), 1)
    row8 = jax.lax.broadcasted_iota(jnp.int32, (H, KV), 0)
    sn = jnp.where(col8 == (row8 >> 2), s_new, NEG)
    m2 = jnp.maximum(m_f, jnp.max(sn, axis=1, keepdims=True))
    corr = jnp.exp(m_f - m2)
    pn = jnp.sum(jnp.exp(sn - m2[:, :1]), axis=1, keepdims=True)
    l2 = l_f * corr + pn
    vn_rep = jnp.repeat(vn_ref[0], G, axis=0)
    out_ref[0, 0] = (acc_f * corr + pn * vn_rep) / l2


def _paged_attn(q, k, v, k_cache, v_cache, block_tables, context_lens,
                interpret=False):
    last_page = (context_lens - 1) // BS
    page_pos = jnp.minimum(jnp.arange(MB, dtype=jnp.int32)[None, :],
                           last_page[:, None])
    phys = jnp.take_along_axis(block_tables, page_pos, axis=1)

    grid_spec = pltpu.PrefetchScalarGridSpec(
        num_scalar_prefetch=2,
        grid=(B,),
        in_specs=[
            pl.BlockSpec((1, H, HD), lambda b, phys, ctx: (b, 0, 0)),
            pl.BlockSpec((1, KV, HD), lambda b, phys, ctx: (b, 0, 0)),
            pl.BlockSpec((1, KV, HD), lambda b, phys, ctx: (b, 0, 0)),
            pl.BlockSpec(memory_space=pl.ANY),
            pl.BlockSpec(memory_space=pl.ANY),
        ],
        out_specs=pl.BlockSpec((1, 1, H, HD),
                               lambda b, phys, ctx: (b, 0, 0, 0)),
        scratch_shapes=[
            pltpu.VMEM((2 * CPI + 2, BPI, BS, KV, HD), jnp.float32),
            pltpu.VMEM((2 * CPI + 2, BPI, BS, KV, HD), jnp.float32),
            pltpu.VMEM((H, LANES), jnp.float32),
            pltpu.VMEM((H, LANES), jnp.int32),
            pltpu.SemaphoreType.DMA((2 * CPI + 2,)),
            pltpu.SemaphoreType.DMA((2 * CPI + 2,)),
        ],
    )
    out = pl.pallas_call(
        _attn_kernel,
        grid_spec=grid_spec,
        out_shape=jax.ShapeDtypeStruct((B, 1, H, HD), jnp.float32),
        compiler_params=pltpu.CompilerParams(
            dimension_semantics=("arbitrary",),
        ),
        name="paged_decode_attn",
        interpret=interpret,
    )(phys, context_lens, q * SCALE, k, v, k_cache, v_cache)
    return out


def kernel(q, k, v, k_cache, v_cache, slot_mapping, block_tables, context_lens):
    del slot_mapping
    return _paged_attn(q, k, v, k_cache, v_cache, block_tables, context_lens)
